# Initial kernel scaffold; baseline (speedup 1.0000x reference)
import functools

import jax
import jax.numpy as jnp
from jax import lax
from jax.experimental import pallas as pl
from jax.experimental.pallas import tpu as pltpu
from jax.experimental.pallas import tpu_sc as plsc

N_NODES = 10000
VOCAB = 21128
EMB_DIM = 128
OUT_DIM = 10
L = 16
NCORE = 2
NSUB = 16
NP = 10240
RPW = NP // NSUB
NYC = RPW // 128


def _sc_mesh():
    return plsc.VectorSubcoreMesh(
        core_axis_name="c", subcore_axis_name="s",
        num_cores=NCORE, num_subcores=NSUB)


def _tc_matmul(emb, w16):
    def body(a_ref, w_ref, o_ref):
        o_ref[...] = jnp.dot(a_ref[...], w_ref[...],
                             preferred_element_type=jnp.float32)
    return pl.pallas_call(
        body,
        out_shape=jax.ShapeDtypeStruct((VOCAB, L), jnp.float32),
    )(emb, w16)


def _sc_gather_and_degree(tw, nidx, dsts, n_chunks):
    @functools.partial(
        pl.kernel,
        out_type=[jax.ShapeDtypeStruct((NCORE, NP, L), jnp.float32),
                  jax.ShapeDtypeStruct((NCORE, NP), jnp.float32)],
        mesh=_sc_mesh(),
        scratch_types=[pltpu.VMEM_SHARED((NP,), jnp.float32),
                       pltpu.VMEM((NYC, 128), jnp.int32),
                       pltpu.VMEM((RPW, L), jnp.float32),
                       pltpu.VMEM((n_chunks, 128), jnp.int32),
                       pltpu.VMEM((RPW,), jnp.float32),
                       pltpu.SemaphoreType.DMA],
    )
    def k(tw_hbm, nidx_hbm, dst_hbm, y_hbm, deg_hbm,
          deg_sh, nidx_v, yrows_v, dst_v, ones_v, sem):
        cid = lax.axis_index("c")
        sid = lax.axis_index("s")
        base = sid * RPW

        @pl.loop(0, RPW, step=L)
        def _(i):
            ones_v[pl.ds(i, L)] = jnp.full((L,), 1.0, jnp.float32)

        pltpu.sync_copy(ones_v, deg_sh.at[pl.ds(base, RPW)])
        pltpu.sync_copy(nidx_hbm.at[cid, sid], nidx_v)
        pltpu.sync_copy(dst_hbm.at[cid, sid], dst_v)
        plsc.subcore_barrier()

        @pl.loop(0, NYC)
        def _(j):
            pltpu.async_copy(tw_hbm.at[nidx_v.at[j]],
                             yrows_v.at[pl.ds(j * 128, 128)], sem).wait()
        pltpu.sync_copy(yrows_v, y_hbm.at[cid, pl.ds(base, RPW)])

        @pl.loop(0, n_chunks)
        def _(j):
            pltpu.sync_copy(ones_v.at[pl.ds(0, 128)],
                            deg_sh.at[dst_v.at[j]], add=True)
        plsc.subcore_barrier()

        pltpu.sync_copy(deg_sh.at[pl.ds(base, RPW)], ones_v)
        pltpu.sync_copy(ones_v, deg_hbm.at[cid, pl.ds(base, RPW)])

    return k(tw, nidx, dsts)


def _tc_dinv_z(deg2, y2):
    def body(deg_ref, y_ref, z_ref, dinv_ref):
        dinv = lax.rsqrt(deg_ref[...])
        dinv_ref[...] = dinv
        z_ref[...] = y_ref[...] * dinv
    return pl.pallas_call(
        body,
        out_shape=[jax.ShapeDtypeStruct((NCORE * NP, L), jnp.float32),
                   jax.ShapeDtypeStruct((NCORE * NP, 1), jnp.float32)],
    )(deg2, y2)


def _sc_scatter(z2, srcs, dsts, n_chunks):
    @functools.partial(
        pl.kernel,
        out_type=jax.ShapeDtypeStruct((NCORE, NP, L), jnp.float32),
        mesh=_sc_mesh(),
        scratch_types=[pltpu.VMEM_SHARED((NP, L), jnp.float32),
                       pltpu.VMEM((n_chunks, 128), jnp.int32),
                       pltpu.VMEM((n_chunks, 128), jnp.int32),
                       pltpu.VMEM((128, L), jnp.float32),
                       pltpu.VMEM((RPW, L), jnp.float32),
                       pltpu.SemaphoreType.DMA],
    )
    def k(z_hbm, src_hbm, dst_hbm, acc_hbm,
          acc_sh, src_v, dst_v, rows_v, stage_v, sem):
        cid = lax.axis_index("c")
        sid = lax.axis_index("s")
        base = sid * RPW

        @pl.loop(0, RPW)
        def _(i):
            stage_v.at[i][...] = jnp.zeros((L,), jnp.float32)
        pltpu.sync_copy(stage_v, acc_sh.at[pl.ds(base, RPW)])
        pltpu.sync_copy(src_hbm.at[cid, sid], src_v)
        pltpu.sync_copy(dst_hbm.at[cid, sid], dst_v)
        plsc.subcore_barrier()

        @pl.loop(0, n_chunks)
        def _(j):
            pltpu.async_copy(z_hbm.at[src_v.at[j]], rows_v, sem).wait()
            pltpu.sync_copy(rows_v, acc_sh.at[dst_v.at[j]], add=True)
        plsc.subcore_barrier()

        pltpu.sync_copy(acc_sh.at[pl.ds(base, RPW)], stage_v)
        pltpu.sync_copy(stage_v, acc_hbm.at[cid, pl.ds(base, RPW)])

    return k(z2, srcs, dsts)


def _tc_combine(acc2, y2, dinv2, b16):
    def body(acc_ref, y_ref, dinv_ref, b_ref, o_ref):
        dv = dinv_ref[...]
        o_ref[...] = dv * acc_ref[...] + (dv * dv) * y_ref[...] + b_ref[...]
    return pl.pallas_call(
        body,
        out_shape=jax.ShapeDtypeStruct((NCORE * NP, L), jnp.float32),
    )(acc2, y2, dinv2, b16)


def kernel(utterance_input, response_input, utterance_graph_adj,
           response_graph_adj, emb_table, W, b):
    e = utterance_graph_adj.shape[1]
    epw = ((e + NSUB * 128 - 1) // (NSUB * 128)) * 128
    n_chunks = epw // 128
    ep = epw * NSUB
    pad = ep - e

    def prep_nodes(idx):
        idx = jnp.concatenate(
            [idx.astype(jnp.int32), jnp.zeros((NP - N_NODES,), jnp.int32)])
        return idx.reshape(NSUB, NYC, 128)

    def prep_edges(adj, gid):
        fill = jnp.full((pad,), NP - 1, jnp.int32)
        src = jnp.concatenate([adj[0].astype(jnp.int32), fill]) + gid * NP
        dst = jnp.concatenate([adj[1].astype(jnp.int32), fill])
        return (src.reshape(NSUB, n_chunks, 128),
                dst.reshape(NSUB, n_chunks, 128))

    nidx = jnp.stack([prep_nodes(utterance_input),
                      prep_nodes(response_input)])
    su, du = prep_edges(utterance_graph_adj, 0)
    sr, dr = prep_edges(response_graph_adj, 1)
    srcs = jnp.stack([su, sr])
    dsts = jnp.stack([du, dr])

    w16 = jnp.pad(W, ((0, 0), (0, L - OUT_DIM)))
    b16 = jnp.pad(b, (0, L - OUT_DIM)).reshape(1, L)

    tw = _tc_matmul(emb_table, w16)
    y, deg = _sc_gather_and_degree(tw, nidx, dsts, n_chunks)
    y2 = y.reshape(NCORE * NP, L)
    z2, dinv2 = _tc_dinv_z(deg.reshape(NCORE * NP, 1), y2)
    acc = _sc_scatter(z2, srcs, dsts, n_chunks)
    out = _tc_combine(acc.reshape(NCORE * NP, L), y2, dinv2, b16)
    out = out.reshape(NCORE, NP, L)
    return (out[0, :N_NODES, :OUT_DIM], out[1, :N_NODES, :OUT_DIM])


# speedup vs baseline: 43.6860x; 43.6860x over previous
import functools

import jax
import jax.numpy as jnp
from jax import lax
from jax.experimental import pallas as pl
from jax.experimental.pallas import tpu as pltpu
from jax.experimental.pallas import tpu_sc as plsc

N_NODES = 10000
VOCAB = 21128
EMB_DIM = 128
OUT_DIM = 10
L = 16
NCORE = 2
NSUB = 16
NP = 10240
RPW = NP // NSUB
NYC = RPW // 128


def _sc_mesh():
    return plsc.VectorSubcoreMesh(
        core_axis_name="c", subcore_axis_name="s",
        num_cores=NCORE, num_subcores=NSUB)


_SC_PARAMS = pltpu.CompilerParams(use_tc_tiling_on_sc=False)


def _tc_matmul(emb, w16):
    def body(a_ref, w_ref, o_ref):
        o_ref[...] = jnp.dot(a_ref[...], w_ref[...],
                             preferred_element_type=jnp.float32)
    return pl.pallas_call(
        body,
        out_shape=jax.ShapeDtypeStruct((VOCAB, L), jnp.float32),
    )(emb, w16)


def _sc_gather_and_degree(tw, nidx, dsts, n_chunks):
    @functools.partial(
        pl.kernel,
        out_type=[jax.ShapeDtypeStruct((NCORE, NP, L), jnp.float32),
                  jax.ShapeDtypeStruct((NCORE, NP), jnp.float32)],
        mesh=_sc_mesh(),
        scratch_types=[pltpu.VMEM_SHARED((NP,), jnp.float32),
                       pltpu.VMEM((NYC, 128), jnp.int32),
                       pltpu.VMEM((RPW, L), jnp.float32),
                       pltpu.VMEM((n_chunks, 128), jnp.int32),
                       pltpu.VMEM((RPW,), jnp.float32),
                       pltpu.SemaphoreType.DMA],
        compiler_params=_SC_PARAMS,
    )
    def k(tw_hbm, nidx_hbm, dst_hbm, y_hbm, deg_hbm,
          deg_sh, nidx_v, yrows_v, dst_v, ones_v, sem):
        cid = lax.axis_index("c")
        sid = lax.axis_index("s")
        base = sid * RPW

        @pl.loop(0, RPW, step=L)
        def _(i):
            ones_v[pl.ds(i, L)] = jnp.full((L,), 1.0, jnp.float32)

        pltpu.sync_copy(ones_v, deg_sh.at[pl.ds(base, RPW)])
        pltpu.sync_copy(nidx_hbm.at[cid, sid], nidx_v)
        pltpu.sync_copy(dst_hbm.at[cid, sid], dst_v)
        plsc.subcore_barrier()

        @pl.loop(0, NYC)
        def _(j):
            pltpu.async_copy(tw_hbm.at[nidx_v.at[j]],
                             yrows_v.at[pl.ds(j * 128, 128)], sem).wait()
        pltpu.sync_copy(yrows_v, y_hbm.at[cid, pl.ds(base, RPW)])

        @pl.loop(0, n_chunks)
        def _(j):
            pltpu.sync_copy(ones_v.at[pl.ds(0, 128)],
                            deg_sh.at[dst_v.at[j]], add=True)
        plsc.subcore_barrier()

        pltpu.sync_copy(deg_sh.at[pl.ds(base, RPW)], ones_v)
        pltpu.sync_copy(ones_v, deg_hbm.at[cid, pl.ds(base, RPW)])

    return k(tw, nidx, dsts)


def _tc_dinv_z(deg2, y2):
    def body(deg_ref, y_ref, z_ref, dinv_ref):
        dinv = lax.rsqrt(deg_ref[...])
        dinv_ref[...] = dinv
        z_ref[...] = y_ref[...] * dinv
    return pl.pallas_call(
        body,
        out_shape=[jax.ShapeDtypeStruct((NCORE * NP, L), jnp.float32),
                   jax.ShapeDtypeStruct((NCORE * NP, 1), jnp.float32)],
    )(deg2, y2)


def _sc_scatter(z2, srcs, dsts, n_chunks):
    @functools.partial(
        pl.kernel,
        out_type=jax.ShapeDtypeStruct((NCORE, NP, L), jnp.float32),
        mesh=_sc_mesh(),
        scratch_types=[pltpu.VMEM_SHARED((NP, L), jnp.float32),
                       pltpu.VMEM((n_chunks, 128), jnp.int32),
                       pltpu.VMEM((n_chunks, 128), jnp.int32),
                       pltpu.VMEM((128, L), jnp.float32),
                       pltpu.VMEM((RPW, L), jnp.float32),
                       pltpu.SemaphoreType.DMA],
        compiler_params=_SC_PARAMS,
    )
    def k(z_hbm, src_hbm, dst_hbm, acc_hbm,
          acc_sh, src_v, dst_v, rows_v, stage_v, sem):
        cid = lax.axis_index("c")
        sid = lax.axis_index("s")
        base = sid * RPW

        @pl.loop(0, RPW)
        def _(i):
            stage_v.at[i][...] = jnp.zeros((L,), jnp.float32)
        pltpu.sync_copy(stage_v, acc_sh.at[pl.ds(base, RPW)])
        pltpu.sync_copy(src_hbm.at[cid, sid], src_v)
        pltpu.sync_copy(dst_hbm.at[cid, sid], dst_v)
        plsc.subcore_barrier()

        @pl.loop(0, n_chunks)
        def _(j):
            pltpu.async_copy(z_hbm.at[src_v.at[j]], rows_v, sem).wait()
            pltpu.sync_copy(rows_v, acc_sh.at[dst_v.at[j]], add=True)
        plsc.subcore_barrier()

        pltpu.sync_copy(acc_sh.at[pl.ds(base, RPW)], stage_v)
        pltpu.sync_copy(stage_v, acc_hbm.at[cid, pl.ds(base, RPW)])

    return k(z2, srcs, dsts)


def _tc_combine(acc2, y2, dinv2, b16):
    def body(acc_ref, y_ref, dinv_ref, b_ref, o_ref):
        dv = dinv_ref[...]
        o_ref[...] = dv * acc_ref[...] + (dv * dv) * y_ref[...] + b_ref[...]
    return pl.pallas_call(
        body,
        out_shape=jax.ShapeDtypeStruct((NCORE * NP, L), jnp.float32),
    )(acc2, y2, dinv2, b16)


def kernel(utterance_input, response_input, utterance_graph_adj,
           response_graph_adj, emb_table, W, b):
    e = utterance_graph_adj.shape[1]
    epw = ((e + NSUB * 128 - 1) // (NSUB * 128)) * 128
    n_chunks = epw // 128
    ep = epw * NSUB
    pad = ep - e

    def prep_nodes(idx):
        idx = jnp.concatenate(
            [idx.astype(jnp.int32), jnp.zeros((NP - N_NODES,), jnp.int32)])
        return idx.reshape(NSUB, NYC, 128)

    def prep_edges(adj, gid):
        fill = jnp.full((pad,), NP - 1, jnp.int32)
        src = jnp.concatenate([adj[0].astype(jnp.int32), fill]) + gid * NP
        dst = jnp.concatenate([adj[1].astype(jnp.int32), fill])
        return (src.reshape(NSUB, n_chunks, 128),
                dst.reshape(NSUB, n_chunks, 128))

    nidx = jnp.stack([prep_nodes(utterance_input),
                      prep_nodes(response_input)])
    su, du = prep_edges(utterance_graph_adj, 0)
    sr, dr = prep_edges(response_graph_adj, 1)
    srcs = jnp.stack([su, sr])
    dsts = jnp.stack([du, dr])

    w16 = jnp.pad(W, ((0, 0), (0, L - OUT_DIM)))
    b16 = jnp.pad(b, (0, L - OUT_DIM)).reshape(1, L)

    tw = _tc_matmul(emb_table, w16)
    y, deg = _sc_gather_and_degree(tw, nidx, dsts, n_chunks)
    y2 = y.reshape(NCORE * NP, L)
    z2, dinv2 = _tc_dinv_z(deg.reshape(NCORE * NP, 1), y2)
    acc = _sc_scatter(z2, srcs, dsts, n_chunks)
    out = _tc_combine(acc.reshape(NCORE * NP, L), y2, dinv2, b16)
    out = out.reshape(NCORE, NP, L)
    return (out[0, :N_NODES, :OUT_DIM], out[1, :N_NODES, :OUT_DIM])


# speedup vs baseline: 55.8523x; 1.2785x over previous
import functools

import jax
import jax.numpy as jnp
from jax import lax
from jax.experimental import pallas as pl
from jax.experimental.pallas import tpu as pltpu
from jax.experimental.pallas import tpu_sc as plsc

N_NODES = 10000
VOCAB = 21128
EMB_DIM = 128
OUT_DIM = 10
L = 16
NCORE = 2
NSUB = 16
NP = 10240
RPW = NP // NSUB
NYC = RPW // 128
NB = 8


def _sc_mesh():
    return plsc.VectorSubcoreMesh(
        core_axis_name="c", subcore_axis_name="s",
        num_cores=NCORE, num_subcores=NSUB)


_SC_PARAMS = pltpu.CompilerParams(use_tc_tiling_on_sc=False)


def _tc_matmul(emb, w16):
    def body(a_ref, w_ref, o_ref):
        o_ref[...] = jnp.dot(a_ref[...], w_ref[...],
                             preferred_element_type=jnp.float32)
    return pl.pallas_call(
        body,
        out_shape=jax.ShapeDtypeStruct((VOCAB, L), jnp.float32),
    )(emb, w16)


def _sc_gather_and_degree(tw, nidx, dsts, n_chunks):
    @functools.partial(
        pl.kernel,
        out_type=[jax.ShapeDtypeStruct((NCORE, NP, L), jnp.float32),
                  jax.ShapeDtypeStruct((NCORE, NP), jnp.float32)],
        mesh=_sc_mesh(),
        scratch_types=[pltpu.VMEM_SHARED((NP,), jnp.float32),
                       pltpu.VMEM((NYC, 128), jnp.int32),
                       pltpu.VMEM((RPW, L), jnp.float32),
                       pltpu.VMEM((n_chunks, 128), jnp.int32),
                       pltpu.VMEM((RPW,), jnp.float32),
                       pltpu.SemaphoreType.DMA],
        compiler_params=_SC_PARAMS,
    )
    def k(tw_hbm, nidx_hbm, dst_hbm, y_hbm, deg_hbm,
          deg_sh, nidx_v, yrows_v, dst_v, ones_v, sem):
        cid = lax.axis_index("c")
        sid = lax.axis_index("s")
        base = sid * RPW

        @pl.loop(0, RPW, step=L)
        def _(i):
            ones_v[pl.ds(i, L)] = jnp.full((L,), 1.0, jnp.float32)

        pltpu.sync_copy(ones_v, deg_sh.at[pl.ds(base, RPW)])
        pltpu.sync_copy(nidx_hbm.at[cid, sid], nidx_v)
        pltpu.sync_copy(dst_hbm.at[cid, sid], dst_v)
        plsc.subcore_barrier()

        gd = [pltpu.async_copy(tw_hbm.at[nidx_v.at[j]],
                               yrows_v.at[pl.ds(j * 128, 128)], sem)
              for j in range(NYC)]
        for d in gd:
            d.wait()
        pltpu.sync_copy(yrows_v, y_hbm.at[cid, pl.ds(base, RPW)])

        @pl.loop(0, n_chunks, step=NB)
        def _(j0):
            dd = [pltpu.async_copy(ones_v.at[pl.ds(0, 128)],
                                   deg_sh.at[dst_v.at[j0 + t]], sem, add=True)
                  for t in range(NB)]
            for d in dd:
                d.wait()
        plsc.subcore_barrier()

        pltpu.sync_copy(deg_sh.at[pl.ds(base, RPW)], ones_v)
        pltpu.sync_copy(ones_v, deg_hbm.at[cid, pl.ds(base, RPW)])

    return k(tw, nidx, dsts)


def _tc_dinv_z(deg2, y2):
    def body(deg_ref, y_ref, z_ref, dinv_ref):
        dinv = lax.rsqrt(deg_ref[...])
        dinv_ref[...] = dinv
        z_ref[...] = y_ref[...] * dinv
    return pl.pallas_call(
        body,
        out_shape=[jax.ShapeDtypeStruct((NCORE * NP, L), jnp.float32),
                   jax.ShapeDtypeStruct((NCORE * NP, 1), jnp.float32)],
    )(deg2, y2)


def _sc_scatter(z2, srcs, dsts, n_chunks):
    @functools.partial(
        pl.kernel,
        out_type=jax.ShapeDtypeStruct((NCORE, NP, L), jnp.float32),
        mesh=_sc_mesh(),
        scratch_types=[pltpu.VMEM_SHARED((NP, L), jnp.float32),
                       pltpu.VMEM((n_chunks, 128), jnp.int32),
                       pltpu.VMEM((n_chunks, 128), jnp.int32),
                       pltpu.VMEM((NB, 128, L), jnp.float32),
                       pltpu.VMEM((RPW, L), jnp.float32),
                       pltpu.SemaphoreType.DMA,
                       pltpu.SemaphoreType.DMA],
        compiler_params=_SC_PARAMS,
    )
    def k(z_hbm, src_hbm, dst_hbm, acc_hbm,
          acc_sh, src_v, dst_v, rows_v, stage_v, gsem, ssem):
        cid = lax.axis_index("c")
        sid = lax.axis_index("s")
        base = sid * RPW

        @pl.loop(0, RPW)
        def _(i):
            stage_v.at[i][...] = jnp.zeros((L,), jnp.float32)
        pltpu.sync_copy(stage_v, acc_sh.at[pl.ds(base, RPW)])
        pltpu.sync_copy(src_hbm.at[cid, sid], src_v)
        pltpu.sync_copy(dst_hbm.at[cid, sid], dst_v)
        plsc.subcore_barrier()

        @pl.loop(0, n_chunks, step=NB)
        def _(j0):
            gd = [pltpu.async_copy(z_hbm.at[src_v.at[j0 + t]],
                                   rows_v.at[t], gsem)
                  for t in range(NB)]
            sd = []
            for t in range(NB):
                gd[t].wait()
                sd.append(pltpu.async_copy(rows_v.at[t],
                                           acc_sh.at[dst_v.at[j0 + t]],
                                           ssem, add=True))
            for d in sd:
                d.wait()
        plsc.subcore_barrier()

        pltpu.sync_copy(acc_sh.at[pl.ds(base, RPW)], stage_v)
        pltpu.sync_copy(stage_v, acc_hbm.at[cid, pl.ds(base, RPW)])

    return k(z2, srcs, dsts)


def _tc_combine(acc2, y2, dinv2, b16):
    def body(acc_ref, y_ref, dinv_ref, b_ref, o_ref):
        dv = dinv_ref[...]
        o_ref[...] = dv * acc_ref[...] + (dv * dv) * y_ref[...] + b_ref[...]
    return pl.pallas_call(
        body,
        out_shape=jax.ShapeDtypeStruct((NCORE * NP, L), jnp.float32),
    )(acc2, y2, dinv2, b16)


def kernel(utterance_input, response_input, utterance_graph_adj,
           response_graph_adj, emb_table, W, b):
    e = utterance_graph_adj.shape[1]
    gran = 128 * NB
    epw = ((e + NSUB * gran - 1) // (NSUB * gran)) * gran
    n_chunks = epw // 128
    ep = epw * NSUB
    pad = ep - e

    def prep_nodes(idx):
        idx = jnp.concatenate(
            [idx.astype(jnp.int32), jnp.zeros((NP - N_NODES,), jnp.int32)])
        return idx.reshape(NSUB, NYC, 128)

    def prep_edges(adj, gid):
        fill = jnp.full((pad,), NP - 1, jnp.int32)
        src = jnp.concatenate([adj[0].astype(jnp.int32), fill]) + gid * NP
        dst = jnp.concatenate([adj[1].astype(jnp.int32), fill])
        return (src.reshape(NSUB, n_chunks, 128),
                dst.reshape(NSUB, n_chunks, 128))

    nidx = jnp.stack([prep_nodes(utterance_input),
                      prep_nodes(response_input)])
    su, du = prep_edges(utterance_graph_adj, 0)
    sr, dr = prep_edges(response_graph_adj, 1)
    srcs = jnp.stack([su, sr])
    dsts = jnp.stack([du, dr])

    w16 = jnp.pad(W, ((0, 0), (0, L - OUT_DIM)))
    b16 = jnp.pad(b, (0, L - OUT_DIM)).reshape(1, L)

    tw = _tc_matmul(emb_table, w16)
    y, deg = _sc_gather_and_degree(tw, nidx, dsts, n_chunks)
    y2 = y.reshape(NCORE * NP, L)
    z2, dinv2 = _tc_dinv_z(deg.reshape(NCORE * NP, 1), y2)
    acc = _sc_scatter(z2, srcs, dsts, n_chunks)
    out = _tc_combine(acc.reshape(NCORE * NP, L), y2, dinv2, b16)
    out = out.reshape(NCORE, NP, L)
    return (out[0, :N_NODES, :OUT_DIM], out[1, :N_NODES, :OUT_DIM])


# speedup vs baseline: 86.3591x; 1.5462x over previous
import functools

import jax
import jax.numpy as jnp
from jax import lax
from jax.experimental import pallas as pl
from jax.experimental.pallas import tpu as pltpu
from jax.experimental.pallas import tpu_sc as plsc

N_NODES = 10000
VOCAB = 21128
EMB_DIM = 128
OUT_DIM = 10
L = 16
NCORE = 2
NSUB = 16
NP = 10240
RPW = NP // NSUB
NYC = RPW // 128
NB = 8

_SC_PARAMS = pltpu.CompilerParams(use_tc_tiling_on_sc=False,
                                  needs_layout_passes=False)


def _tc_matmul(emb, w16):
    def body(a_ref, w_ref, o_ref):
        o_ref[...] = jnp.dot(a_ref[...], w_ref[...],
                             preferred_element_type=jnp.float32)
    return pl.pallas_call(
        body,
        out_shape=jax.ShapeDtypeStruct((VOCAB, L), jnp.float32),
    )(emb, w16)


def _rsqrt16(x):
    i = plsc.bitcast(x, jnp.int32)
    h = plsc.bitcast(jnp.full((L,), 0x5F3759DF, jnp.int32)
                     - (i >> jnp.full((L,), 1, jnp.int32)), jnp.float32)
    for _ in range(3):
        h = h * (1.5 - 0.5 * x * h * h)
    return h


def _sc_gcn(tw, uidx, ridx, su, du, sr, dr, b16, epr, n_chunks):
    epw = n_chunks * 128

    @functools.partial(
        pl.kernel,
        out_type=jax.ShapeDtypeStruct((NCORE, NP, L), jnp.float32),
        mesh=plsc.VectorSubcoreMesh(
            core_axis_name="c", subcore_axis_name="s",
            num_cores=NCORE, num_subcores=NSUB),
        scratch_types=[pltpu.VMEM_SHARED((NP,), jnp.float32),
                       pltpu.VMEM_SHARED((NP, L), jnp.float32),
                       pltpu.VMEM_SHARED((NP, L), jnp.float32),
                       pltpu.VMEM((RPW,), jnp.int32),
                       pltpu.VMEM((epw,), jnp.int32),
                       pltpu.VMEM((epw,), jnp.int32),
                       pltpu.VMEM((RPW, L), jnp.float32),
                       pltpu.VMEM((RPW, L), jnp.float32),
                       pltpu.VMEM((RPW, L), jnp.float32),
                       pltpu.VMEM((NB, 128, L), jnp.float32),
                       pltpu.VMEM((RPW,), jnp.float32),
                       pltpu.VMEM((RPW,), jnp.float32),
                       pltpu.VMEM((L,), jnp.float32),
                       pltpu.SemaphoreType.DMA,
                       pltpu.SemaphoreType.DMA,
                       pltpu.SemaphoreType.DMA],
        compiler_params=_SC_PARAMS,
    )
    def k(tw_hbm, uidx_hbm, ridx_hbm, su_hbm, du_hbm, sr_hbm, dr_hbm, b_hbm,
          out_hbm, deg_sh, z_sh, acc_sh,
          nidx_v, src_v, dst_v, yrows_v, stage_v, dinv16_v, rows_v,
          ones_v, dinv_v, b_v, gsem, ssem, hsem):
        cid = lax.axis_index("c")
        sid = lax.axis_index("s")
        base = sid * RPW
        ebase = sid * epr

        @pl.when(cid == 0)
        def _():
            pltpu.async_copy(uidx_hbm.at[pl.ds(base, RPW)], nidx_v, hsem)
            pltpu.async_copy(su_hbm.at[pl.ds(ebase, epr)],
                             src_v.at[pl.ds(0, epr)], hsem)
            pltpu.async_copy(du_hbm.at[pl.ds(ebase, epr)],
                             dst_v.at[pl.ds(0, epr)], hsem)

        @pl.when(cid == 1)
        def _():
            pltpu.async_copy(ridx_hbm.at[pl.ds(base, RPW)], nidx_v, hsem)
            pltpu.async_copy(sr_hbm.at[pl.ds(ebase, epr)],
                             src_v.at[pl.ds(0, epr)], hsem)
            pltpu.async_copy(dr_hbm.at[pl.ds(ebase, epr)],
                             dst_v.at[pl.ds(0, epr)], hsem)
        pltpu.async_copy(b_hbm, b_v, hsem)

        @pl.loop(0, RPW, step=L)
        def _(i):
            ones_v[pl.ds(i, L)] = jnp.full((L,), 1.0, jnp.float32)

        @pl.loop(0, RPW)
        def _(i):
            stage_v.at[i][...] = jnp.zeros((L,), jnp.float32)

        @pl.loop(epr, epw, step=L)
        def _(i):
            src_v[pl.ds(i, L)] = jnp.full((L,), NP - 1, jnp.int32)
            dst_v[pl.ds(i, L)] = jnp.full((L,), NP - 1, jnp.int32)

        for _ in range(4):
            pltpu.make_async_copy(b_hbm, b_v, hsem).wait()
        pltpu.sync_copy(ones_v, deg_sh.at[pl.ds(base, RPW)])
        pltpu.sync_copy(stage_v, acc_sh.at[pl.ds(base, RPW)])

        gd = [pltpu.async_copy(tw_hbm.at[nidx_v.at[pl.ds(j * 128, 128)]],
                               yrows_v.at[pl.ds(j * 128, 128)], gsem)
              for j in range(NYC)]
        plsc.subcore_barrier()

        @pl.loop(0, n_chunks, step=NB)
        def _(j0):
            dd = [pltpu.async_copy(ones_v.at[pl.ds(0, 128)],
                                   deg_sh.at[dst_v.at[pl.ds((j0 + t) * 128,
                                                            128)]],
                                   ssem, add=True)
                  for t in range(NB)]
            for d in dd:
                d.wait()
        for d in gd:
            d.wait()
        plsc.subcore_barrier()

        pltpu.sync_copy(deg_sh.at[pl.ds(base, RPW)], ones_v)

        @pl.loop(0, RPW // L, unroll=4)
        def _(i):
            x = ones_v[pl.ds(i * L, L)]
            dinv_v[pl.ds(i * L, L)] = _rsqrt16(x)

        @pl.loop(0, RPW, unroll=4)
        def _(r):
            dv = plsc.load_gather(dinv_v, [jnp.full((L,), r, jnp.int32)])
            dinv16_v.at[r][...] = dv
            yrows_v.at[r][...] = yrows_v.at[r][...] * dv

        pltpu.sync_copy(yrows_v, z_sh.at[pl.ds(base, RPW)])
        plsc.subcore_barrier()

        @pl.loop(0, n_chunks, step=NB)
        def _(j0):
            gds = [pltpu.async_copy(
                z_sh.at[src_v.at[pl.ds((j0 + t) * 128, 128)]],
                rows_v.at[t], gsem) for t in range(NB)]
            sds = []
            for t in range(NB):
                gds[t].wait()
                sds.append(pltpu.async_copy(
                    rows_v.at[t],
                    acc_sh.at[dst_v.at[pl.ds((j0 + t) * 128, 128)]],
                    ssem, add=True))
            for d in sds:
                d.wait()
        plsc.subcore_barrier()

        pltpu.sync_copy(acc_sh.at[pl.ds(base, RPW)], stage_v)
        bvec = b_v[...]

        @pl.loop(0, RPW, unroll=4)
        def _(r):
            stage_v.at[r][...] = (dinv16_v.at[r][...]
                                  * (stage_v.at[r][...] + yrows_v.at[r][...])
                                  + bvec)
        pltpu.sync_copy(stage_v, out_hbm.at[cid, pl.ds(base, RPW)])

    return k(tw, uidx, ridx, su, du, sr, dr, b16)


def kernel(utterance_input, response_input, utterance_graph_adj,
           response_graph_adj, emb_table, W, b):
    e = utterance_graph_adj.shape[1]
    epr = e // NSUB
    gran = 128 * NB
    n_chunks = ((epr + gran - 1) // gran) * NB

    npad = NP - N_NODES
    uidx = jnp.concatenate([utterance_input.astype(jnp.int32),
                            jnp.zeros((npad,), jnp.int32)])
    ridx = jnp.concatenate([response_input.astype(jnp.int32),
                            jnp.zeros((npad,), jnp.int32)])
    su = utterance_graph_adj[0].astype(jnp.int32)
    du = utterance_graph_adj[1].astype(jnp.int32)
    sr = response_graph_adj[0].astype(jnp.int32)
    dr = response_graph_adj[1].astype(jnp.int32)

    w16 = jnp.pad(W, ((0, 0), (0, L - OUT_DIM)))
    b16 = jnp.pad(b, (0, L - OUT_DIM))

    tw = _tc_matmul(emb_table, w16)
    out = _sc_gcn(tw, uidx, ridx, su, du, sr, dr, b16, epr, n_chunks)
    return (out[0, :N_NODES, :OUT_DIM], out[1, :N_NODES, :OUT_DIM])


# speedup vs baseline: 93.4105x; 1.0817x over previous
import functools

import jax
import jax.numpy as jnp
from jax import lax
from jax.experimental import pallas as pl
from jax.experimental.pallas import tpu as pltpu
from jax.experimental.pallas import tpu_sc as plsc

N_NODES = 10000
VOCAB = 21128
EMB_DIM = 128
OUT_DIM = 10
L = 16
NCORE = 2
NSUB = 16
NP = 10240
RPW = NP // NSUB
NYC = RPW // 128
NB = 8

_SC_PARAMS = pltpu.CompilerParams(use_tc_tiling_on_sc=False,
                                  needs_layout_passes=False)


def _tc_matmul(emb, w16):
    blk = 1112
    def body(a_ref, w_ref, o_ref):
        o_ref[...] = jnp.dot(a_ref[...], w_ref[...],
                             preferred_element_type=jnp.float32)
    return pl.pallas_call(
        body,
        grid=(VOCAB // blk,),
        in_specs=[pl.BlockSpec((blk, EMB_DIM), lambda i: (i, 0)),
                  pl.BlockSpec((EMB_DIM, L), lambda i: (0, 0))],
        out_specs=pl.BlockSpec((blk, L), lambda i: (i, 0)),
        out_shape=jax.ShapeDtypeStruct((VOCAB, L), jnp.float32),
    )(emb, w16)


def _rsqrt16(x):
    i = plsc.bitcast(x, jnp.int32)
    h = plsc.bitcast(jnp.full((L,), 0x5F3759DF, jnp.int32)
                     - (i >> jnp.full((L,), 1, jnp.int32)), jnp.float32)
    for _ in range(3):
        h = h * (1.5 - 0.5 * x * h * h)
    return h


def _sc_gcn(tw, uidx, ridx, adj_u, adj_r, b16, epr, n_chunks):
    epw = n_chunks * 128

    @functools.partial(
        pl.kernel,
        out_type=jax.ShapeDtypeStruct((NCORE, NP, L), jnp.float32),
        mesh=plsc.VectorSubcoreMesh(
            core_axis_name="c", subcore_axis_name="s",
            num_cores=NCORE, num_subcores=NSUB),
        scratch_types=[pltpu.VMEM_SHARED((NP,), jnp.float32),
                       pltpu.VMEM_SHARED((NP, L), jnp.float32),
                       pltpu.VMEM_SHARED((NP, L), jnp.float32),
                       pltpu.VMEM((RPW,), jnp.int32),
                       pltpu.VMEM((epw,), jnp.int32),
                       pltpu.VMEM((epw,), jnp.int32),
                       pltpu.VMEM((RPW, L), jnp.float32),
                       pltpu.VMEM((RPW, L), jnp.float32),
                       pltpu.VMEM((RPW, L), jnp.float32),
                       pltpu.VMEM((NB, 128, L), jnp.float32),
                       pltpu.VMEM((RPW,), jnp.float32),
                       pltpu.VMEM((RPW,), jnp.float32),
                       pltpu.VMEM((L,), jnp.float32),
                       pltpu.SemaphoreType.DMA,
                       pltpu.SemaphoreType.DMA,
                       pltpu.SemaphoreType.DMA],
        compiler_params=_SC_PARAMS,
    )
    def k(tw_hbm, uidx_hbm, ridx_hbm, adju_hbm, adjr_hbm, b_hbm,
          out_hbm, deg_sh, z_sh, acc_sh,
          nidx_v, src_v, dst_v, yrows_v, stage_v, dinv16_v, rows_v,
          ones_v, dinv_v, b_v, gsem, ssem, hsem):
        cid = lax.axis_index("c")
        sid = lax.axis_index("s")
        base = sid * RPW
        ebase = sid * epr

        @pl.when(cid == 0)
        def _():
            pltpu.async_copy(uidx_hbm.at[pl.ds(base, RPW)], nidx_v, hsem)
            pltpu.async_copy(adju_hbm.at[0].at[pl.ds(ebase, epr)],
                             src_v.at[pl.ds(0, epr)], hsem)
            pltpu.async_copy(adju_hbm.at[1].at[pl.ds(ebase, epr)],
                             dst_v.at[pl.ds(0, epr)], hsem)

        @pl.when(cid == 1)
        def _():
            pltpu.async_copy(ridx_hbm.at[pl.ds(base, RPW)], nidx_v, hsem)
            pltpu.async_copy(adjr_hbm.at[0].at[pl.ds(ebase, epr)],
                             src_v.at[pl.ds(0, epr)], hsem)
            pltpu.async_copy(adjr_hbm.at[1].at[pl.ds(ebase, epr)],
                             dst_v.at[pl.ds(0, epr)], hsem)
        pltpu.async_copy(b_hbm, b_v, hsem)

        @pl.loop(0, RPW, step=L)
        def _(i):
            ones_v[pl.ds(i, L)] = jnp.full((L,), 1.0, jnp.float32)

        @pl.loop(0, RPW)
        def _(i):
            stage_v.at[i][...] = jnp.zeros((L,), jnp.float32)

        @pl.loop(epr, epw, step=L)
        def _(i):
            src_v[pl.ds(i, L)] = jnp.full((L,), NP - 1, jnp.int32)
            dst_v[pl.ds(i, L)] = jnp.full((L,), NP - 1, jnp.int32)

        for _ in range(4):
            pltpu.make_async_copy(b_hbm, b_v, hsem).wait()
        pltpu.sync_copy(ones_v, deg_sh.at[pl.ds(base, RPW)])
        pltpu.sync_copy(stage_v, acc_sh.at[pl.ds(base, RPW)])

        gd = [pltpu.async_copy(tw_hbm.at[nidx_v.at[pl.ds(j * 128, 128)]],
                               yrows_v.at[pl.ds(j * 128, 128)], gsem)
              for j in range(NYC)]
        plsc.subcore_barrier()

        @pl.loop(0, n_chunks, step=NB)
        def _(j0):
            dd = [pltpu.async_copy(ones_v.at[pl.ds(0, 128)],
                                   deg_sh.at[dst_v.at[pl.ds((j0 + t) * 128,
                                                            128)]],
                                   ssem, add=True)
                  for t in range(NB)]
            for d in dd:
                d.wait()
        for d in gd:
            d.wait()
        plsc.subcore_barrier()

        pltpu.sync_copy(deg_sh.at[pl.ds(base, RPW)], ones_v)

        @pl.loop(0, RPW // L, unroll=4)
        def _(i):
            x = ones_v[pl.ds(i * L, L)]
            dinv_v[pl.ds(i * L, L)] = _rsqrt16(x)

        @pl.loop(0, RPW, unroll=4)
        def _(r):
            dv = plsc.load_gather(dinv_v, [jnp.full((L,), r, jnp.int32)])
            dinv16_v.at[r][...] = dv
            yrows_v.at[r][...] = yrows_v.at[r][...] * dv

        pltpu.sync_copy(yrows_v, z_sh.at[pl.ds(base, RPW)])
        plsc.subcore_barrier()

        @pl.loop(0, n_chunks, step=NB)
        def _(j0):
            gds = [pltpu.async_copy(
                z_sh.at[src_v.at[pl.ds((j0 + t) * 128, 128)]],
                rows_v.at[t], gsem) for t in range(NB)]
            sds = []
            for t in range(NB):
                gds[t].wait()
                sds.append(pltpu.async_copy(
                    rows_v.at[t],
                    acc_sh.at[dst_v.at[pl.ds((j0 + t) * 128, 128)]],
                    ssem, add=True))
            for d in sds:
                d.wait()
        plsc.subcore_barrier()

        pltpu.sync_copy(acc_sh.at[pl.ds(base, RPW)], stage_v)
        bvec = b_v[...]

        @pl.loop(0, RPW, unroll=4)
        def _(r):
            stage_v.at[r][...] = (dinv16_v.at[r][...]
                                  * (stage_v.at[r][...] + yrows_v.at[r][...])
                                  + bvec)
        pltpu.sync_copy(stage_v, out_hbm.at[cid, pl.ds(base, RPW)])

    return k(tw, uidx, ridx, adj_u, adj_r, b16)


def kernel(utterance_input, response_input, utterance_graph_adj,
           response_graph_adj, emb_table, W, b):
    e = utterance_graph_adj.shape[1]
    epr = e // NSUB
    gran = 128 * NB
    n_chunks = ((epr + gran - 1) // gran) * NB

    npad = NP - N_NODES
    uidx = jnp.concatenate([utterance_input.astype(jnp.int32),
                            jnp.zeros((npad,), jnp.int32)])
    ridx = jnp.concatenate([response_input.astype(jnp.int32),
                            jnp.zeros((npad,), jnp.int32)])
    w16 = jnp.pad(W, ((0, 0), (0, L - OUT_DIM)))
    b16 = jnp.pad(b, (0, L - OUT_DIM))

    tw = _tc_matmul(emb_table, w16)
    out = _sc_gcn(tw, uidx, ridx, utterance_graph_adj,
                  response_graph_adj, b16, epr, n_chunks)
    return (out[0, :N_NODES, :OUT_DIM], out[1, :N_NODES, :OUT_DIM])


# speedup vs baseline: 108.7475x; 1.1642x over previous
import functools

import jax
import jax.numpy as jnp
from jax import lax
from jax.experimental import pallas as pl
from jax.experimental.pallas import tpu as pltpu
from jax.experimental.pallas import tpu_sc as plsc

N_NODES = 10000
VOCAB = 21128
EMB_DIM = 128
OUT_DIM = 10
L = 16
NCORE = 2
NSUB = 16
NP = 10240
RPW = NP // NSUB
NYC = RPW // 128
NB = 8

_SC_PARAMS = pltpu.CompilerParams(use_tc_tiling_on_sc=False,
                                  needs_layout_passes=False)

_MESH = dict(core_axis_name="c", subcore_axis_name="s",
             num_cores=NCORE, num_subcores=NSUB)


def _tc_matmul(emb, w16):
    def body(a_ref, w_ref, o_ref):
        o_ref[...] = jnp.dot(a_ref[...], w_ref[...],
                             preferred_element_type=jnp.float32)
    return pl.pallas_call(
        body,
        out_shape=jax.ShapeDtypeStruct((VOCAB, L), jnp.float32),
    )(emb, w16)


def _rsqrt16(x):
    i = plsc.bitcast(x, jnp.int32)
    h = plsc.bitcast(jnp.full((L,), 0x5F3759DF, jnp.int32)
                     - (i >> jnp.full((L,), 1, jnp.int32)), jnp.float32)
    for _ in range(3):
        h = h * (1.5 - 0.5 * x * h * h)
    return h


def _sc_degree(adj_u, adj_r, epr, n_chunks):
    epw = n_chunks * 128

    @functools.partial(
        pl.kernel,
        out_type=jax.ShapeDtypeStruct((NCORE, NP), jnp.float32),
        mesh=plsc.VectorSubcoreMesh(**_MESH),
        scratch_types=[pltpu.VMEM_SHARED((NP,), jnp.float32),
                       pltpu.VMEM((epw,), jnp.int32),
                       pltpu.VMEM((RPW,), jnp.float32),
                       pltpu.SemaphoreType.DMA,
                       pltpu.SemaphoreType.DMA],
        compiler_params=_SC_PARAMS,
    )
    def k(adju_hbm, adjr_hbm, deg_hbm, deg_sh, dst_v, ones_v, ssem, hsem):
        cid = lax.axis_index("c")
        sid = lax.axis_index("s")
        base = sid * RPW
        ebase = sid * epr

        @pl.when(cid == 0)
        def _():
            pltpu.async_copy(adju_hbm.at[1].at[pl.ds(ebase, epr)],
                             dst_v.at[pl.ds(0, epr)], hsem)

        @pl.when(cid == 1)
        def _():
            pltpu.async_copy(adjr_hbm.at[1].at[pl.ds(ebase, epr)],
                             dst_v.at[pl.ds(0, epr)], hsem)

        @pl.loop(0, RPW, step=L)
        def _(i):
            ones_v[pl.ds(i, L)] = jnp.full((L,), 1.0, jnp.float32)

        @pl.loop(epr, epw, step=L)
        def _(i):
            dst_v[pl.ds(i, L)] = jnp.full((L,), NP - 1, jnp.int32)

        pltpu.make_async_copy(adju_hbm.at[1].at[pl.ds(ebase, epr)],
                              dst_v.at[pl.ds(0, epr)], hsem).wait()
        pltpu.sync_copy(ones_v, deg_sh.at[pl.ds(base, RPW)])
        plsc.subcore_barrier()

        @pl.loop(0, n_chunks, step=NB)
        def _(j0):
            dd = [pltpu.async_copy(ones_v.at[pl.ds(0, 128)],
                                   deg_sh.at[dst_v.at[pl.ds((j0 + t) * 128,
                                                            128)]],
                                   ssem, add=True)
                  for t in range(NB)]
            for d in dd:
                d.wait()
        plsc.subcore_barrier()

        pltpu.sync_copy(deg_sh.at[pl.ds(base, RPW)], ones_v)
        pltpu.sync_copy(ones_v, deg_hbm.at[cid, pl.ds(base, RPW)])

    return k(adj_u, adj_r)


def _sc_main(tw, uidx, ridx, adj_u, adj_r, deg, b16, epr, n_chunks):
    epw = n_chunks * 128

    @functools.partial(
        pl.kernel,
        out_type=jax.ShapeDtypeStruct((NCORE, NP, L), jnp.float32),
        mesh=plsc.VectorSubcoreMesh(**_MESH),
        scratch_types=[pltpu.VMEM_SHARED((NP, L), jnp.float32),
                       pltpu.VMEM_SHARED((NP, L), jnp.float32),
                       pltpu.VMEM((RPW,), jnp.int32),
                       pltpu.VMEM((epw,), jnp.int32),
                       pltpu.VMEM((epw,), jnp.int32),
                       pltpu.VMEM((RPW, L), jnp.float32),
                       pltpu.VMEM((RPW, L), jnp.float32),
                       pltpu.VMEM((RPW, L), jnp.float32),
                       pltpu.VMEM((NB, 128, L), jnp.float32),
                       pltpu.VMEM((RPW,), jnp.float32),
                       pltpu.VMEM((RPW,), jnp.float32),
                       pltpu.VMEM((L,), jnp.float32),
                       pltpu.SemaphoreType.DMA,
                       pltpu.SemaphoreType.DMA,
                       pltpu.SemaphoreType.DMA],
        compiler_params=_SC_PARAMS,
    )
    def k(tw_hbm, uidx_hbm, ridx_hbm, adju_hbm, adjr_hbm, deg_hbm, b_hbm,
          out_hbm, z_sh, acc_sh,
          nidx_v, src_v, dst_v, yrows_v, stage_v, dinv16_v, rows_v,
          deg_v, dinv_v, b_v, gsem, ssem, hsem):
        cid = lax.axis_index("c")
        sid = lax.axis_index("s")
        base = sid * RPW
        ebase = sid * epr

        @pl.when(cid == 0)
        def _():
            pltpu.async_copy(uidx_hbm.at[pl.ds(base, RPW)], nidx_v, hsem)
            pltpu.async_copy(adju_hbm.at[0].at[pl.ds(ebase, epr)],
                             src_v.at[pl.ds(0, epr)], hsem)
            pltpu.async_copy(adju_hbm.at[1].at[pl.ds(ebase, epr)],
                             dst_v.at[pl.ds(0, epr)], hsem)

        @pl.when(cid == 1)
        def _():
            pltpu.async_copy(ridx_hbm.at[pl.ds(base, RPW)], nidx_v, hsem)
            pltpu.async_copy(adjr_hbm.at[0].at[pl.ds(ebase, epr)],
                             src_v.at[pl.ds(0, epr)], hsem)
            pltpu.async_copy(adjr_hbm.at[1].at[pl.ds(ebase, epr)],
                             dst_v.at[pl.ds(0, epr)], hsem)
        pltpu.async_copy(deg_hbm.at[cid, pl.ds(base, RPW)], deg_v, hsem)
        pltpu.async_copy(b_hbm, b_v, hsem)

        @pl.loop(0, RPW)
        def _(i):
            stage_v.at[i][...] = jnp.zeros((L,), jnp.float32)

        @pl.loop(epr, epw, step=L)
        def _(i):
            src_v[pl.ds(i, L)] = jnp.full((L,), NP - 1, jnp.int32)
            dst_v[pl.ds(i, L)] = jnp.full((L,), NP - 1, jnp.int32)

        pltpu.make_async_copy(uidx_hbm.at[pl.ds(base, RPW)],
                              nidx_v, hsem).wait()
        pltpu.make_async_copy(adju_hbm.at[0].at[pl.ds(ebase, epr)],
                              src_v.at[pl.ds(0, epr)], hsem).wait()
        pltpu.make_async_copy(adju_hbm.at[1].at[pl.ds(ebase, epr)],
                              dst_v.at[pl.ds(0, epr)], hsem).wait()
        pltpu.make_async_copy(deg_hbm.at[cid, pl.ds(base, RPW)],
                              deg_v, hsem).wait()
        pltpu.make_async_copy(b_hbm, b_v, hsem).wait()

        pltpu.sync_copy(stage_v, acc_sh.at[pl.ds(base, RPW)])

        gd = [pltpu.async_copy(tw_hbm.at[nidx_v.at[pl.ds(j * 128, 128)]],
                               yrows_v.at[pl.ds(j * 128, 128)], gsem)
              for j in range(NYC)]

        @pl.loop(0, RPW // L, unroll=4)
        def _(i):
            dinv_v[pl.ds(i * L, L)] = _rsqrt16(deg_v[pl.ds(i * L, L)])

        for d in gd:
            d.wait()

        @pl.loop(0, RPW, unroll=4)
        def _(r):
            dv = plsc.load_gather(dinv_v, [jnp.full((L,), r, jnp.int32)])
            dinv16_v.at[r][...] = dv
            yrows_v.at[r][...] = yrows_v.at[r][...] * dv

        pltpu.sync_copy(yrows_v, z_sh.at[pl.ds(base, RPW)])
        plsc.subcore_barrier()

        @pl.loop(0, n_chunks, step=NB)
        def _(j0):
            gds = [pltpu.async_copy(
                z_sh.at[src_v.at[pl.ds((j0 + t) * 128, 128)]],
                rows_v.at[t], gsem) for t in range(NB)]
            sds = []
            for t in range(NB):
                gds[t].wait()
                sds.append(pltpu.async_copy(
                    rows_v.at[t],
                    acc_sh.at[dst_v.at[pl.ds((j0 + t) * 128, 128)]],
                    ssem, add=True))
            for d in sds:
                d.wait()
        plsc.subcore_barrier()

        pltpu.sync_copy(acc_sh.at[pl.ds(base, RPW)], stage_v)
        bvec = b_v[...]

        @pl.loop(0, RPW, unroll=4)
        def _(r):
            stage_v.at[r][...] = (dinv16_v.at[r][...]
                                  * (stage_v.at[r][...] + yrows_v.at[r][...])
                                  + bvec)
        pltpu.sync_copy(stage_v, out_hbm.at[cid, pl.ds(base, RPW)])

    return k(tw, uidx, ridx, adj_u, adj_r, deg, b16)


def kernel(utterance_input, response_input, utterance_graph_adj,
           response_graph_adj, emb_table, W, b):
    e = utterance_graph_adj.shape[1]
    epr = e // NSUB
    gran = 128 * NB
    n_chunks = ((epr + gran - 1) // gran) * NB

    npad = NP - N_NODES
    uidx = jnp.concatenate([utterance_input.astype(jnp.int32),
                            jnp.zeros((npad,), jnp.int32)])
    ridx = jnp.concatenate([response_input.astype(jnp.int32),
                            jnp.zeros((npad,), jnp.int32)])
    w16 = jnp.pad(W, ((0, 0), (0, L - OUT_DIM)))
    b16 = jnp.pad(b, (0, L - OUT_DIM))

    deg = _sc_degree(utterance_graph_adj, response_graph_adj, epr, n_chunks)
    tw = _tc_matmul(emb_table, w16)
    out = _sc_main(tw, uidx, ridx, utterance_graph_adj, response_graph_adj,
                   deg, b16, epr, n_chunks)
    return (out[0, :N_NODES, :OUT_DIM], out[1, :N_NODES, :OUT_DIM])


# speedup vs baseline: 112.8795x; 1.0380x over previous
import functools

import jax
import jax.numpy as jnp
from jax import lax
from jax.experimental import pallas as pl
from jax.experimental.pallas import tpu as pltpu
from jax.experimental.pallas import tpu_sc as plsc

N_NODES = 10000
VOCAB = 21128
EMB_DIM = 128
OUT_DIM = 10
L = 16
NCORE = 2
NSUB = 16
NP = 10240
RPW = NP // NSUB
NYC = RPW // 128
NB = 16

_SC_PARAMS = pltpu.CompilerParams(use_tc_tiling_on_sc=False,
                                  needs_layout_passes=False)

_MESH = dict(core_axis_name="c", subcore_axis_name="s",
             num_cores=NCORE, num_subcores=NSUB)


def _tc_matmul(emb, w16):
    def body(a_ref, w_ref, o_ref):
        o_ref[...] = jnp.dot(a_ref[...], w_ref[...],
                             preferred_element_type=jnp.float32)
    return pl.pallas_call(
        body,
        out_shape=jax.ShapeDtypeStruct((VOCAB, L), jnp.float32),
    )(emb, w16)


def _rsqrt16(x):
    i = plsc.bitcast(x, jnp.int32)
    h = plsc.bitcast(jnp.full((L,), 0x5F3759DF, jnp.int32)
                     - (i >> jnp.full((L,), 1, jnp.int32)), jnp.float32)
    for _ in range(3):
        h = h * (1.5 - 0.5 * x * h * h)
    return h


def _sc_degree(adj_u, adj_r, epr, n_chunks):
    epw = n_chunks * 128

    @functools.partial(
        pl.kernel,
        out_type=jax.ShapeDtypeStruct((NCORE, NP), jnp.float32),
        mesh=plsc.VectorSubcoreMesh(**_MESH),
        scratch_types=[pltpu.VMEM_SHARED((NP,), jnp.float32),
                       pltpu.VMEM((epw,), jnp.int32),
                       pltpu.VMEM((RPW,), jnp.float32),
                       pltpu.SemaphoreType.DMA,
                       pltpu.SemaphoreType.DMA],
        compiler_params=_SC_PARAMS,
    )
    def k(adju_hbm, adjr_hbm, deg_hbm, deg_sh, dst_v, ones_v, ssem, hsem):
        cid = lax.axis_index("c")
        sid = lax.axis_index("s")
        base = sid * RPW
        ebase = sid * epr

        @pl.when(cid == 0)
        def _():
            pltpu.async_copy(adju_hbm.at[1].at[pl.ds(ebase, epr)],
                             dst_v.at[pl.ds(0, epr)], hsem)

        @pl.when(cid == 1)
        def _():
            pltpu.async_copy(adjr_hbm.at[1].at[pl.ds(ebase, epr)],
                             dst_v.at[pl.ds(0, epr)], hsem)

        @pl.loop(0, RPW, step=L)
        def _(i):
            ones_v[pl.ds(i, L)] = jnp.full((L,), 1.0, jnp.float32)

        @pl.loop(epr, epw, step=L)
        def _(i):
            dst_v[pl.ds(i, L)] = jnp.full((L,), NP - 1, jnp.int32)

        pltpu.make_async_copy(adju_hbm.at[1].at[pl.ds(ebase, epr)],
                              dst_v.at[pl.ds(0, epr)], hsem).wait()
        pltpu.sync_copy(ones_v, deg_sh.at[pl.ds(base, RPW)])
        plsc.subcore_barrier()

        @pl.loop(0, n_chunks, step=NB)
        def _(j0):
            dd = [pltpu.async_copy(ones_v.at[pl.ds(0, 128)],
                                   deg_sh.at[dst_v.at[pl.ds((j0 + t) * 128,
                                                            128)]],
                                   ssem, add=True)
                  for t in range(NB)]
            for d in dd:
                d.wait()
        plsc.subcore_barrier()

        pltpu.sync_copy(deg_sh.at[pl.ds(base, RPW)], ones_v)
        pltpu.sync_copy(ones_v, deg_hbm.at[cid, pl.ds(base, RPW)])

    return k(adj_u, adj_r)


def _sc_main(tw, uidx, ridx, adj_u, adj_r, deg, b16, epr, n_chunks):
    epw = n_chunks * 128

    @functools.partial(
        pl.kernel,
        out_type=jax.ShapeDtypeStruct((NCORE, NP, L), jnp.float32),
        mesh=plsc.VectorSubcoreMesh(**_MESH),
        scratch_types=[pltpu.VMEM_SHARED((NP, L), jnp.float32),
                       pltpu.VMEM_SHARED((NP, L), jnp.float32),
                       pltpu.VMEM((RPW,), jnp.int32),
                       pltpu.VMEM((epw,), jnp.int32),
                       pltpu.VMEM((epw,), jnp.int32),
                       pltpu.VMEM((RPW, L), jnp.float32),
                       pltpu.VMEM((RPW, L), jnp.float32),
                       pltpu.VMEM((RPW, L), jnp.float32),
                       pltpu.VMEM((NB, 128, L), jnp.float32),
                       pltpu.VMEM((RPW,), jnp.float32),
                       pltpu.VMEM((RPW,), jnp.float32),
                       pltpu.VMEM((L,), jnp.float32),
                       pltpu.SemaphoreType.DMA,
                       pltpu.SemaphoreType.DMA,
                       pltpu.SemaphoreType.DMA],
        compiler_params=_SC_PARAMS,
    )
    def k(tw_hbm, uidx_hbm, ridx_hbm, adju_hbm, adjr_hbm, deg_hbm, b_hbm,
          out_hbm, z_sh, acc_sh,
          nidx_v, src_v, dst_v, yrows_v, stage_v, dinv16_v, rows_v,
          deg_v, dinv_v, b_v, gsem, ssem, hsem):
        cid = lax.axis_index("c")
        sid = lax.axis_index("s")
        base = sid * RPW
        ebase = sid * epr

        @pl.when(cid == 0)
        def _():
            pltpu.async_copy(uidx_hbm.at[pl.ds(base, RPW)], nidx_v, hsem)
            pltpu.async_copy(adju_hbm.at[0].at[pl.ds(ebase, epr)],
                             src_v.at[pl.ds(0, epr)], hsem)
            pltpu.async_copy(adju_hbm.at[1].at[pl.ds(ebase, epr)],
                             dst_v.at[pl.ds(0, epr)], hsem)

        @pl.when(cid == 1)
        def _():
            pltpu.async_copy(ridx_hbm.at[pl.ds(base, RPW)], nidx_v, hsem)
            pltpu.async_copy(adjr_hbm.at[0].at[pl.ds(ebase, epr)],
                             src_v.at[pl.ds(0, epr)], hsem)
            pltpu.async_copy(adjr_hbm.at[1].at[pl.ds(ebase, epr)],
                             dst_v.at[pl.ds(0, epr)], hsem)
        pltpu.async_copy(deg_hbm.at[cid, pl.ds(base, RPW)], deg_v, hsem)
        pltpu.async_copy(b_hbm, b_v, hsem)

        @pl.loop(0, RPW)
        def _(i):
            stage_v.at[i][...] = jnp.zeros((L,), jnp.float32)

        @pl.loop(epr, epw, step=L)
        def _(i):
            src_v[pl.ds(i, L)] = jnp.full((L,), NP - 1, jnp.int32)
            dst_v[pl.ds(i, L)] = jnp.full((L,), NP - 1, jnp.int32)

        pltpu.make_async_copy(uidx_hbm.at[pl.ds(base, RPW)],
                              nidx_v, hsem).wait()
        pltpu.make_async_copy(adju_hbm.at[0].at[pl.ds(ebase, epr)],
                              src_v.at[pl.ds(0, epr)], hsem).wait()
        pltpu.make_async_copy(adju_hbm.at[1].at[pl.ds(ebase, epr)],
                              dst_v.at[pl.ds(0, epr)], hsem).wait()
        pltpu.make_async_copy(deg_hbm.at[cid, pl.ds(base, RPW)],
                              deg_v, hsem).wait()
        pltpu.make_async_copy(b_hbm, b_v, hsem).wait()

        pltpu.sync_copy(stage_v, acc_sh.at[pl.ds(base, RPW)])

        gd = [pltpu.async_copy(tw_hbm.at[nidx_v.at[pl.ds(j * 128, 128)]],
                               yrows_v.at[pl.ds(j * 128, 128)], gsem)
              for j in range(NYC)]

        @pl.loop(0, RPW // L, unroll=4)
        def _(i):
            dinv_v[pl.ds(i * L, L)] = _rsqrt16(deg_v[pl.ds(i * L, L)])

        for d in gd:
            d.wait()

        @pl.loop(0, RPW, unroll=4)
        def _(r):
            dv = plsc.load_gather(dinv_v, [jnp.full((L,), r, jnp.int32)])
            dinv16_v.at[r][...] = dv
            yrows_v.at[r][...] = yrows_v.at[r][...] * dv

        pltpu.sync_copy(yrows_v, z_sh.at[pl.ds(base, RPW)])
        plsc.subcore_barrier()

        @pl.loop(0, n_chunks, step=NB)
        def _(j0):
            gds = [pltpu.async_copy(
                z_sh.at[src_v.at[pl.ds((j0 + t) * 128, 128)]],
                rows_v.at[t], gsem) for t in range(NB)]
            sds = []
            for t in range(NB):
                gds[t].wait()
                sds.append(pltpu.async_copy(
                    rows_v.at[t],
                    acc_sh.at[dst_v.at[pl.ds((j0 + t) * 128, 128)]],
                    ssem, add=True))
            for d in sds:
                d.wait()
        plsc.subcore_barrier()

        pltpu.sync_copy(acc_sh.at[pl.ds(base, RPW)], stage_v)
        bvec = b_v[...]

        @pl.loop(0, RPW, unroll=4)
        def _(r):
            stage_v.at[r][...] = (dinv16_v.at[r][...]
                                  * (stage_v.at[r][...] + yrows_v.at[r][...])
                                  + bvec)
        pltpu.sync_copy(stage_v, out_hbm.at[cid, pl.ds(base, RPW)])

    return k(tw, uidx, ridx, adj_u, adj_r, deg, b16)


def kernel(utterance_input, response_input, utterance_graph_adj,
           response_graph_adj, emb_table, W, b):
    e = utterance_graph_adj.shape[1]
    epr = e // NSUB
    gran = 128 * NB
    n_chunks = ((epr + gran - 1) // gran) * NB

    npad = NP - N_NODES
    uidx = jnp.concatenate([utterance_input.astype(jnp.int32),
                            jnp.zeros((npad,), jnp.int32)])
    ridx = jnp.concatenate([response_input.astype(jnp.int32),
                            jnp.zeros((npad,), jnp.int32)])
    w16 = jnp.pad(W, ((0, 0), (0, L - OUT_DIM)))
    b16 = jnp.pad(b, (0, L - OUT_DIM))

    deg = _sc_degree(utterance_graph_adj, response_graph_adj, epr, n_chunks)
    tw = _tc_matmul(emb_table, w16)
    out = _sc_main(tw, uidx, ridx, utterance_graph_adj, response_graph_adj,
                   deg, b16, epr, n_chunks)
    return (out[0, :N_NODES, :OUT_DIM], out[1, :N_NODES, :OUT_DIM])
